# Initial kernel scaffold; baseline (speedup 1.0000x reference)
import jax
import jax.numpy as jnp
from jax.experimental import pallas as pl


def kernel(item_seq, item_seq_len, emb_table, Wa, ba, Wb, bb):
    raise NotImplementedError("write your pallas kernel here")


# speedup vs baseline: 1.3986x; 1.3986x over previous
import functools

import jax
import jax.numpy as jnp
from jax import lax
from jax.experimental import pallas as pl
from jax.experimental.pallas import tpu as pltpu
from jax.experimental.pallas import tpu_sc as plsc

B = 4096
L = 200
D = 8
NC = 2
NS = 16
NW = NC * NS
RPW = B // NW
S1 = 128
S2 = L - S1
NV1 = S1 * D // 16
NV2 = S2 * D // 16

_mesh = plsc.VectorSubcoreMesh(
    core_axis_name="c", subcore_axis_name="s", num_cores=NC, num_subcores=NS
)


@functools.partial(
    pl.kernel,
    mesh=_mesh,
    out_type=(
        jax.ShapeDtypeStruct((B * 16,), jnp.float32),
        jax.ShapeDtypeStruct((B, D), jnp.float32),
    ),
    scratch_types=(
        pltpu.VMEM((RPW, L), jnp.int32),
        pltpu.VMEM((RPW,), jnp.int32),
        pltpu.VMEM((RPW,), jnp.int32),
        pltpu.VMEM((RPW, D), jnp.float32),
        pltpu.VMEM((S1, D), jnp.float32),
        pltpu.VMEM((S2, D), jnp.float32),
        pltpu.VMEM((RPW * 16,), jnp.float32),
        pltpu.SemaphoreType.DMA,
        pltpu.SemaphoreType.DMA,
    ),
    compiler_params=pltpu.CompilerParams(
        needs_layout_passes=False, use_tc_tiling_on_sc=False),
)
def _sc_gather_sum(seq_hbm, len_hbm, table_hbm,
                   ms_out, mt_out,
                   idxblk, len_v, lastid_v, lastrow_v,
                   bufA, bufB, stage, sem, sem2):
    wid = lax.axis_index("s") * NC + lax.axis_index("c")
    base = wid * RPW

    pltpu.sync_copy(seq_hbm.at[pl.ds(base, RPW)], idxblk)
    pltpu.sync_copy(len_hbm.at[pl.ds(base, RPW)], len_v)

    lane = lax.iota(jnp.int32, 16)

    for k in range(RPW // 16):
        lv = len_v[pl.ds(k * 16, 16)]
        rows = (k * 16) + lane
        lastid_v[pl.ds(k * 16, 16)] = plsc.load_gather(
            idxblk, [rows, lv - 1])
    pltpu.async_copy(table_hbm.at[lastid_v], lastrow_v, sem).wait()
    pltpu.sync_copy(lastrow_v, mt_out.at[pl.ds(base, RPW)])

    rowpat = lane >> 3
    colpat = lane & 7

    def row_body(r, carry):
        cp1 = pltpu.async_copy(table_hbm.at[idxblk.at[r, pl.ds(0, S1)]],
                               bufA, sem)
        cp2 = pltpu.async_copy(table_hbm.at[idxblk.at[r, pl.ds(S1, S2)]],
                               bufB, sem2)
        cp1.wait()
        accs = [jnp.zeros((16,), jnp.float32) for _ in range(4)]
        for j in range(NV1):
            accs[j & 3] = accs[j & 3] + plsc.load_gather(
                bufA, [rowpat + 2 * j, colpat])
        cp2.wait()
        for j in range(NV2):
            accs[j & 3] = accs[j & 3] + plsc.load_gather(
                bufB, [rowpat + 2 * j, colpat])
        acc = (accs[0] + accs[1]) + (accs[2] + accs[3])
        stage[pl.ds(r * 16, 16)] = acc
        return carry

    lax.fori_loop(0, RPW, row_body, 0)

    pltpu.sync_copy(stage, ms_out.at[pl.ds(base * 16, RPW * 16)])


def _tc_tail(st_ref, lenf_ref, mt_ref, wat_ref, ba_ref, wbt_ref, bb_ref,
             out_ref):
    st = st_ref[...]
    ms = (st[:, :D] + st[:, D:]) / lenf_ref[...]
    hs = jnp.tanh(
        jnp.dot(ms, wat_ref[...], preferred_element_type=jnp.float32)
        + ba_ref[...])
    ht = jnp.tanh(
        jnp.dot(mt_ref[...], wbt_ref[...], preferred_element_type=jnp.float32)
        + bb_ref[...])
    out_ref[...] = hs * ht


_tc_call = pl.pallas_call(
    _tc_tail,
    out_shape=jax.ShapeDtypeStruct((B, D), jnp.float32),
)


def kernel(item_seq, item_seq_len, emb_table, Wa, ba, Wb, bb):
    item_seq = item_seq.astype(jnp.int32)
    lens = item_seq_len.astype(jnp.int32)
    stage_flat, mt = _sc_gather_sum(item_seq, lens, emb_table)
    stage2d = stage_flat.reshape(B, 16)
    lenf = lens.astype(jnp.float32).reshape(B, 1)
    return _tc_call(stage2d, lenf, mt, Wa.T, ba.reshape(1, D), Wb.T,
                    bb.reshape(1, D))


# speedup vs baseline: 1.5801x; 1.1298x over previous
import functools

import jax
import jax.numpy as jnp
from jax import lax
from jax.experimental import pallas as pl
from jax.experimental.pallas import tpu as pltpu
from jax.experimental.pallas import tpu_sc as plsc

B = 4096
L = 200
D = 8
NC = 2
NS = 16
NW = NC * NS
RPW = B // NW
S1 = 128
S2 = L - S1
NV1 = S1 * D // 16
NV2 = S2 * D // 16
NB = 4

_mesh = plsc.VectorSubcoreMesh(
    core_axis_name="c", subcore_axis_name="s", num_cores=NC, num_subcores=NS
)


@functools.partial(
    pl.kernel,
    mesh=_mesh,
    out_type=(
        jax.ShapeDtypeStruct((B * 16,), jnp.float32),
        jax.ShapeDtypeStruct((B, D), jnp.float32),
    ),
    scratch_types=(
        pltpu.VMEM((RPW, L), jnp.int32),
        pltpu.VMEM((RPW,), jnp.int32),
        pltpu.VMEM((RPW,), jnp.int32),
        pltpu.VMEM((RPW, D), jnp.float32),
        [pltpu.VMEM((S1, D), jnp.float32) for _ in range(4)],
        [pltpu.VMEM((S2, D), jnp.float32) for _ in range(4)],
        pltpu.VMEM((RPW * 16,), jnp.float32),
        [pltpu.SemaphoreType.DMA for _ in range(4)],
    ),
    compiler_params=pltpu.CompilerParams(
        needs_layout_passes=False, use_tc_tiling_on_sc=False),
)
def _sc_gather_sum(seq_hbm, len_hbm, table_hbm,
                   ms_out, mt_out,
                   idxblk, len_v, lastid_v, lastrow_v,
                   bufAs, bufBs, stage, sems):
    wid = lax.axis_index("s") * NC + lax.axis_index("c")
    base = wid * RPW

    pltpu.sync_copy(seq_hbm.at[pl.ds(base, RPW)], idxblk)
    pltpu.sync_copy(len_hbm.at[pl.ds(base, RPW)], len_v)

    lane = lax.iota(jnp.int32, 16)

    for k in range(RPW // 16):
        lv = len_v[pl.ds(k * 16, 16)]
        rows = (k * 16) + lane
        lastid_v[pl.ds(k * 16, 16)] = plsc.load_gather(
            idxblk, [rows, lv - 1])
    pltpu.async_copy(table_hbm.at[lastid_v], lastrow_v, sems[0]).wait()
    pltpu.sync_copy(lastrow_v, mt_out.at[pl.ds(base, RPW)])

    rowpat = lane >> 3
    colpat = lane & 7

    def issue(row, b):
        cp1 = pltpu.async_copy(table_hbm.at[idxblk.at[row, pl.ds(0, S1)]],
                               bufAs[b], sems[b])
        cp2 = pltpu.async_copy(table_hbm.at[idxblk.at[row, pl.ds(S1, S2)]],
                               bufBs[b], sems[b])
        return cp1, cp2

    for b in range(NB):
        issue(b, b)

    def wait_set(b):
        pltpu.make_async_copy(
            table_hbm.at[idxblk.at[0, pl.ds(0, S1)]], bufAs[b], sems[b]
        ).wait()
        pltpu.make_async_copy(
            table_hbm.at[idxblk.at[0, pl.ds(S1, S2)]], bufBs[b], sems[b]
        ).wait()

    @pl.loop(0, RPW, step=NB)
    def row_loop(r):
        for b in range(NB):
            rr = r + b
            wait_set(b)
            accs = [jnp.zeros((16,), jnp.float32) for _ in range(4)]
            for j in range(NV1):
                accs[j & 3] = accs[j & 3] + plsc.load_gather(
                    bufAs[b], [rowpat + 2 * j, colpat])
            for j in range(NV2):
                accs[j & 3] = accs[j & 3] + plsc.load_gather(
                    bufBs[b], [rowpat + 2 * j, colpat])
            acc = (accs[0] + accs[1]) + (accs[2] + accs[3])
            stage[pl.ds(rr * 16, 16)] = acc
            issue((rr + NB) % RPW, b)

    for b in range(NB):
        wait_set(b)

    pltpu.sync_copy(stage, ms_out.at[pl.ds(base * 16, RPW * 16)])


def _tc_tail(st_ref, lenf_ref, mt_ref, wat_ref, ba_ref, wbt_ref, bb_ref,
             out_ref):
    st = st_ref[...]
    ms = (st[:, :D] + st[:, D:]) / lenf_ref[...]
    hs = jnp.tanh(
        jnp.dot(ms, wat_ref[...], preferred_element_type=jnp.float32)
        + ba_ref[...])
    ht = jnp.tanh(
        jnp.dot(mt_ref[...], wbt_ref[...], preferred_element_type=jnp.float32)
        + bb_ref[...])
    out_ref[...] = hs * ht


_tc_call = pl.pallas_call(
    _tc_tail,
    out_shape=jax.ShapeDtypeStruct((B, D), jnp.float32),
)


def kernel(item_seq, item_seq_len, emb_table, Wa, ba, Wb, bb):
    item_seq = item_seq.astype(jnp.int32)
    lens = item_seq_len.astype(jnp.int32)
    stage_flat, mt = _sc_gather_sum(item_seq, lens, emb_table)
    stage2d = stage_flat.reshape(B, 16)
    lenf = lens.astype(jnp.float32).reshape(B, 1)
    return _tc_call(stage2d, lenf, mt, Wa.T, ba.reshape(1, D), Wb.T,
                    bb.reshape(1, D))
